# baseline (speedup 1.0000x reference)
import functools

import jax
import jax.numpy as jnp
from jax import lax
from jax.experimental import pallas as pl
from jax.experimental.pallas import tpu as pltpu
from jax.experimental.pallas import tpu_sc as plsc

N_NODES = 10000
N_EDGES = 320000
F_DIM = 128
S_DIM = 16
N_ROUNDS = 4
N_GRAPHS = 64

NC = 2
NS = 16
N_TILES = NC * NS

CHUNK = 128
K_GRP = 8
E_PAD = N_TILES * 80 * CHUNK
N_CHUNKS = E_PAD // CHUNK
CH_A = 48
CH_B = 112
CH_MAX = max(CH_A, CH_B)
N_PAD = 10240
ROWS_PER_TILE = N_PAD // NS

BLK = 1000
N_BLKS = N_NODES // BLK


def _sc_edge_body(msg_hbm, edge_hbm, out_hbm, src_v, dst_v, gbuf, zbuf, agg,
                  sem_i, sem_ga, sem_gb, sem_sa, sem_sb):
    c = lax.axis_index("c")
    s = lax.axis_index("s")
    base = jnp.where(c == 0, s * CH_A, NS * CH_A + s * CH_B)
    n_grps = jnp.where(c == 0, CH_A // K_GRP, CH_B // K_GRP)
    K = K_GRP

    pltpu.async_copy(edge_hbm.at[0, pl.ds(base, CH_MAX)], src_v, sem_i)
    pltpu.async_copy(edge_hbm.at[1, pl.ds(base, CH_MAX)], dst_v, sem_i)

    zrow = jnp.zeros((S_DIM,), jnp.float32)

    def _zb(i, _):
        zbuf[i] = zrow
        return 0

    lax.fori_loop(0, ROWS_PER_TILE, _zb, 0)
    pltpu.sync_copy(zbuf, agg.at[pl.ds(s * ROWS_PER_TILE, ROWS_PER_TILE)])
    pltpu.make_async_copy(edge_hbm.at[0, pl.ds(base, CH_MAX)], src_v,
                          sem_i).wait()
    pltpu.make_async_copy(edge_hbm.at[1, pl.ds(base, CH_MAX)], dst_v,
                          sem_i).wait()

    sem_g = (sem_ga, sem_gb)
    sem_s = (sem_sa, sem_sb)

    def fire_gathers(g, h):
        for b in range(K):
            pltpu.async_copy(msg_hbm.at[src_v.at[g * K + b]], gbuf.at[h * K + b],
                             sem_g[h])

    def wait_gathers(g, h):
        for b in range(K):
            pltpu.make_async_copy(msg_hbm.at[src_v.at[g * K + b]],
                                  gbuf.at[h * K + b], sem_g[h]).wait()

    def fire_scatters(g, h):
        for b in range(K):
            pltpu.async_copy(gbuf.at[h * K + b], agg.at[dst_v.at[g * K + b]],
                             sem_s[h], add=True)

    def drain_scatters(g, h):
        for b in range(K):
            pltpu.make_async_copy(gbuf.at[h * K + b], agg.at[dst_v.at[g * K + b]],
                                  sem_s[h]).wait()

    fire_gathers(0, 0)
    plsc.subcore_barrier()

    def _outer(i, _):
        q = i * 2
        wait_gathers(q, 0)
        fire_scatters(q, 0)

        @pl.when(q > 0)
        def _():
            drain_scatters(q - 1, 1)

        fire_gathers(q + 1, 1)
        wait_gathers(q + 1, 1)
        fire_scatters(q + 1, 1)
        drain_scatters(q, 0)

        @pl.when(q < n_grps - 2)
        def _():
            fire_gathers(q + 2, 0)

        return 0

    lax.fori_loop(0, n_grps // 2, _outer, 0)
    drain_scatters(n_grps - 1, 1)
    plsc.subcore_barrier()

    pltpu.sync_copy(
        agg.at[pl.ds(s * ROWS_PER_TILE, ROWS_PER_TILE)],
        out_hbm.at[c, pl.ds(s * ROWS_PER_TILE, ROWS_PER_TILE)],
    )


_sc_edge = pl.kernel(
    _sc_edge_body,
    out_type=jax.ShapeDtypeStruct((NC, N_PAD, S_DIM), jnp.float32),
    mesh=plsc.VectorSubcoreMesh(core_axis_name="c", subcore_axis_name="s"),
    compiler_params=pltpu.CompilerParams(use_tc_tiling_on_sc=False),
    scratch_types=[
        pltpu.VMEM((CH_MAX, CHUNK), jnp.int32),
        pltpu.VMEM((CH_MAX, CHUNK), jnp.int32),
        pltpu.VMEM((2 * K_GRP, CHUNK, S_DIM), jnp.float32),
        pltpu.VMEM((ROWS_PER_TILE, S_DIM), jnp.float32),
        pltpu.VMEM_SHARED((N_PAD, S_DIM), jnp.float32),
        pltpu.SemaphoreType.DMA,
        pltpu.SemaphoreType.DMA,
        pltpu.SemaphoreType.DMA,
        pltpu.SemaphoreType.DMA,
        pltpu.SemaphoreType.DMA,
    ],
)


def _tc_init_body(x_ref, wi_ref, bi_ref, wm_ref, bm_ref, st_ref, msg_ref):
    st = jnp.maximum(
        jnp.dot(x_ref[...], wi_ref[...], preferred_element_type=jnp.float32)
        + bi_ref[...],
        0.0,
    )
    st_ref[...] = st
    msg_ref[...] = jnp.maximum(
        jnp.dot(st, wm_ref[...], preferred_element_type=jnp.float32) + bm_ref[...],
        0.0,
    )


def _tc_init(x, wi, bi, wm, bm):
    return pl.pallas_call(
        _tc_init_body,
        grid=(N_BLKS,),
        in_specs=[
            pl.BlockSpec((BLK, F_DIM), lambda i: (i, 0)),
            pl.BlockSpec((F_DIM, S_DIM), lambda i: (0, 0)),
            pl.BlockSpec((1, S_DIM), lambda i: (0, 0)),
            pl.BlockSpec((S_DIM, S_DIM), lambda i: (0, 0)),
            pl.BlockSpec((1, S_DIM), lambda i: (0, 0)),
        ],
        out_specs=[
            pl.BlockSpec((BLK, S_DIM), lambda i: (i, 0)),
            pl.BlockSpec((BLK, S_DIM), lambda i: (i, 0)),
        ],
        out_shape=[
            jax.ShapeDtypeStruct((N_NODES, S_DIM), jnp.float32),
            jax.ShapeDtypeStruct((N_NODES, S_DIM), jnp.float32),
        ],
    )(x, wi, bi, wm, bm)


def _tc_upd_body(st_ref, p_ref, wu_ref, bu_ref, wm_ref, bm_ref, st_out, msg_out):
    a = p_ref[0] + p_ref[1]
    st = st_ref[...] + jnp.maximum(
        jnp.dot(a, wu_ref[...], preferred_element_type=jnp.float32) + bu_ref[...],
        0.0,
    )
    st_out[...] = st
    msg_out[...] = jnp.maximum(
        jnp.dot(st, wm_ref[...], preferred_element_type=jnp.float32) + bm_ref[...],
        0.0,
    )


def _tc_upd(st, parts, wu, bu, wm, bm):
    return pl.pallas_call(
        _tc_upd_body,
        grid=(N_BLKS,),
        in_specs=[
            pl.BlockSpec((BLK, S_DIM), lambda i: (i, 0)),
            pl.BlockSpec((NC, BLK, S_DIM), lambda i: (0, i, 0)),
            pl.BlockSpec((S_DIM, S_DIM), lambda i: (0, 0)),
            pl.BlockSpec((1, S_DIM), lambda i: (0, 0)),
            pl.BlockSpec((S_DIM, S_DIM), lambda i: (0, 0)),
            pl.BlockSpec((1, S_DIM), lambda i: (0, 0)),
        ],
        out_specs=[
            pl.BlockSpec((BLK, S_DIM), lambda i: (i, 0)),
            pl.BlockSpec((BLK, S_DIM), lambda i: (i, 0)),
        ],
        out_shape=[
            jax.ShapeDtypeStruct((N_NODES, S_DIM), jnp.float32),
            jax.ShapeDtypeStruct((N_NODES, S_DIM), jnp.float32),
        ],
    )(st, parts, wu, bu, wm, bm)


def _tc_fin_body(st_ref, p_ref, wu_ref, bu_ref, b_ref, wo_ref, bo_ref, out_ref, gs_ref):
    i = pl.program_id(0)

    @pl.when(i == 0)
    def _():
        gs_ref[...] = jnp.zeros_like(gs_ref)

    a = p_ref[0] + p_ref[1]
    st = st_ref[...] + jnp.maximum(
        jnp.dot(a, wu_ref[...], preferred_element_type=jnp.float32) + bu_ref[...],
        0.0,
    )
    b = b_ref[0, 0, :]
    onehot = (
        lax.broadcasted_iota(jnp.int32, (N_GRAPHS, BLK), 0) == b[None, :]
    ).astype(jnp.float32)
    gs_ref[...] += jnp.dot(onehot, st, preferred_element_type=jnp.float32)

    @pl.when(i == pl.num_programs(0) - 1)
    def _():
        out_ref[...] = (
            jnp.dot(gs_ref[...], wo_ref[...], preferred_element_type=jnp.float32)
            + bo_ref[...]
        )


def _tc_fin(st, parts, wu, bu, batch3, wo, bo):
    return pl.pallas_call(
        _tc_fin_body,
        grid=(N_BLKS,),
        in_specs=[
            pl.BlockSpec((BLK, S_DIM), lambda i: (i, 0)),
            pl.BlockSpec((NC, BLK, S_DIM), lambda i: (0, i, 0)),
            pl.BlockSpec((S_DIM, S_DIM), lambda i: (0, 0)),
            pl.BlockSpec((1, S_DIM), lambda i: (0, 0)),
            pl.BlockSpec((1, 1, BLK), lambda i: (i, 0, 0)),
            pl.BlockSpec((S_DIM, 1), lambda i: (0, 0)),
            pl.BlockSpec((1, 1), lambda i: (0, 0)),
        ],
        out_specs=pl.BlockSpec((N_GRAPHS, 1), lambda i: (0, 0)),
        out_shape=jax.ShapeDtypeStruct((N_GRAPHS, 1), jnp.float32),
        scratch_shapes=[pltpu.VMEM((N_GRAPHS, S_DIM), jnp.float32)],
    )(st, parts, wu, bu, batch3, wo, bo)


def kernel(x, edge_index, batch, Wi, bi, Wm, bm, Wu, bu, Wo, bo):
    pad = jnp.concatenate(
        [
            jnp.zeros((1, E_PAD - N_EDGES), jnp.int32),
            jnp.full((1, E_PAD - N_EDGES), N_NODES, jnp.int32),
        ],
        axis=0,
    )
    edge3 = jnp.concatenate([edge_index, pad], axis=1).reshape(2, N_CHUNKS, CHUNK)
    batch3 = batch.reshape(N_BLKS, 1, BLK)
    bi2 = bi.reshape(1, S_DIM)
    bo2 = bo.reshape(1, 1)

    st, msg = _tc_init(x, Wi, bi2, Wm[0], bm[0].reshape(1, S_DIM))
    for r in range(N_ROUNDS):
        parts = _sc_edge(msg, edge3)
        if r < N_ROUNDS - 1:
            st, msg = _tc_upd(
                st, parts,
                Wu[r], bu[r].reshape(1, S_DIM),
                Wm[r + 1], bm[r + 1].reshape(1, S_DIM),
            )
        else:
            out = _tc_fin(st, parts, Wu[r], bu[r].reshape(1, S_DIM), batch3, Wo, bo2)
    return out.reshape(-1)


# speedup vs baseline: 1.1189x; 1.1189x over previous
import functools

import jax
import jax.numpy as jnp
from jax import lax
from jax.experimental import pallas as pl
from jax.experimental.pallas import tpu as pltpu
from jax.experimental.pallas import tpu_sc as plsc

N_NODES = 10000
N_EDGES = 320000
F_DIM = 128
S_DIM = 16
N_ROUNDS = 4
N_GRAPHS = 64

NC = 2
NS = 16
N_TILES = NC * NS

CHUNK = 128
K_GRP = 8
E_PAD = N_TILES * 80 * CHUNK
N_CHUNKS = E_PAD // CHUNK
CH_A = 112
CH_B = 48
CH_MAX = max(CH_A, CH_B)
N_PAD = 10240
ROWS_PER_TILE = N_PAD // NS

BLK = 1000
N_BLKS = N_NODES // BLK


def _sc_edge_body(msg_hbm, edge_hbm, out_hbm, src_v, dst_v, gbuf, zbuf, agg,
                  sem_i, sem_ga, sem_gb, sem_sa, sem_sb):
    c = lax.axis_index("c")
    s = lax.axis_index("s")
    base = jnp.where(c == 0, s * CH_A, NS * CH_A + s * CH_B)
    n_grps = jnp.where(c == 0, CH_A // K_GRP, CH_B // K_GRP)
    K = K_GRP

    pltpu.async_copy(edge_hbm.at[0, pl.ds(base, CH_MAX)], src_v, sem_i)
    pltpu.async_copy(edge_hbm.at[1, pl.ds(base, CH_MAX)], dst_v, sem_i)

    zrow = jnp.zeros((S_DIM,), jnp.float32)

    def _zb(i, _):
        zbuf[i] = zrow
        return 0

    lax.fori_loop(0, ROWS_PER_TILE, _zb, 0)
    pltpu.sync_copy(zbuf, agg.at[pl.ds(s * ROWS_PER_TILE, ROWS_PER_TILE)])
    pltpu.make_async_copy(edge_hbm.at[0, pl.ds(base, CH_MAX)], src_v,
                          sem_i).wait()
    pltpu.make_async_copy(edge_hbm.at[1, pl.ds(base, CH_MAX)], dst_v,
                          sem_i).wait()

    sem_g = (sem_ga, sem_gb)
    sem_s = (sem_sa, sem_sb)

    def fire_gathers(g, h):
        for b in range(K):
            pltpu.async_copy(msg_hbm.at[src_v.at[g * K + b]], gbuf.at[h * K + b],
                             sem_g[h])

    def wait_gathers(g, h):
        for b in range(K):
            pltpu.make_async_copy(msg_hbm.at[src_v.at[g * K + b]],
                                  gbuf.at[h * K + b], sem_g[h]).wait()

    def fire_scatters(g, h):
        for b in range(K):
            pltpu.async_copy(gbuf.at[h * K + b], agg.at[dst_v.at[g * K + b]],
                             sem_s[h], add=True)

    def drain_scatters(g, h):
        for b in range(K):
            pltpu.make_async_copy(gbuf.at[h * K + b], agg.at[dst_v.at[g * K + b]],
                                  sem_s[h]).wait()

    fire_gathers(0, 0)
    plsc.subcore_barrier()

    def _outer(i, _):
        q = i * 2
        wait_gathers(q, 0)
        fire_scatters(q, 0)

        @pl.when(q > 0)
        def _():
            drain_scatters(q - 1, 1)

        fire_gathers(q + 1, 1)
        wait_gathers(q + 1, 1)
        fire_scatters(q + 1, 1)
        drain_scatters(q, 0)

        @pl.when(q < n_grps - 2)
        def _():
            fire_gathers(q + 2, 0)

        return 0

    lax.fori_loop(0, n_grps // 2, _outer, 0)
    drain_scatters(n_grps - 1, 1)
    plsc.subcore_barrier()

    pltpu.sync_copy(
        agg.at[pl.ds(s * ROWS_PER_TILE, ROWS_PER_TILE)],
        out_hbm.at[c, pl.ds(s * ROWS_PER_TILE, ROWS_PER_TILE)],
    )


_sc_edge = pl.kernel(
    _sc_edge_body,
    out_type=jax.ShapeDtypeStruct((NC, N_PAD, S_DIM), jnp.float32),
    mesh=plsc.VectorSubcoreMesh(core_axis_name="c", subcore_axis_name="s"),
    compiler_params=pltpu.CompilerParams(use_tc_tiling_on_sc=False),
    scratch_types=[
        pltpu.VMEM((CH_MAX, CHUNK), jnp.int32),
        pltpu.VMEM((CH_MAX, CHUNK), jnp.int32),
        pltpu.VMEM((2 * K_GRP, CHUNK, S_DIM), jnp.float32),
        pltpu.VMEM((ROWS_PER_TILE, S_DIM), jnp.float32),
        pltpu.VMEM_SHARED((N_PAD, S_DIM), jnp.float32),
        pltpu.SemaphoreType.DMA,
        pltpu.SemaphoreType.DMA,
        pltpu.SemaphoreType.DMA,
        pltpu.SemaphoreType.DMA,
        pltpu.SemaphoreType.DMA,
    ],
)


def _tc_init_body(x_ref, wi_ref, bi_ref, wm_ref, bm_ref, st_ref, msg_ref):
    st = jnp.maximum(
        jnp.dot(x_ref[...], wi_ref[...], preferred_element_type=jnp.float32)
        + bi_ref[...],
        0.0,
    )
    st_ref[...] = st
    msg_ref[...] = jnp.maximum(
        jnp.dot(st, wm_ref[...], preferred_element_type=jnp.float32) + bm_ref[...],
        0.0,
    )


def _tc_init(x, wi, bi, wm, bm):
    return pl.pallas_call(
        _tc_init_body,
        grid=(N_BLKS,),
        in_specs=[
            pl.BlockSpec((BLK, F_DIM), lambda i: (i, 0)),
            pl.BlockSpec((F_DIM, S_DIM), lambda i: (0, 0)),
            pl.BlockSpec((1, S_DIM), lambda i: (0, 0)),
            pl.BlockSpec((S_DIM, S_DIM), lambda i: (0, 0)),
            pl.BlockSpec((1, S_DIM), lambda i: (0, 0)),
        ],
        out_specs=[
            pl.BlockSpec((BLK, S_DIM), lambda i: (i, 0)),
            pl.BlockSpec((BLK, S_DIM), lambda i: (i, 0)),
        ],
        out_shape=[
            jax.ShapeDtypeStruct((N_NODES, S_DIM), jnp.float32),
            jax.ShapeDtypeStruct((N_NODES, S_DIM), jnp.float32),
        ],
    )(x, wi, bi, wm, bm)


def _tc_upd_body(st_ref, p_ref, wu_ref, bu_ref, wm_ref, bm_ref, st_out, msg_out):
    a = p_ref[0] + p_ref[1]
    st = st_ref[...] + jnp.maximum(
        jnp.dot(a, wu_ref[...], preferred_element_type=jnp.float32) + bu_ref[...],
        0.0,
    )
    st_out[...] = st
    msg_out[...] = jnp.maximum(
        jnp.dot(st, wm_ref[...], preferred_element_type=jnp.float32) + bm_ref[...],
        0.0,
    )


def _tc_upd(st, parts, wu, bu, wm, bm):
    return pl.pallas_call(
        _tc_upd_body,
        grid=(N_BLKS,),
        in_specs=[
            pl.BlockSpec((BLK, S_DIM), lambda i: (i, 0)),
            pl.BlockSpec((NC, BLK, S_DIM), lambda i: (0, i, 0)),
            pl.BlockSpec((S_DIM, S_DIM), lambda i: (0, 0)),
            pl.BlockSpec((1, S_DIM), lambda i: (0, 0)),
            pl.BlockSpec((S_DIM, S_DIM), lambda i: (0, 0)),
            pl.BlockSpec((1, S_DIM), lambda i: (0, 0)),
        ],
        out_specs=[
            pl.BlockSpec((BLK, S_DIM), lambda i: (i, 0)),
            pl.BlockSpec((BLK, S_DIM), lambda i: (i, 0)),
        ],
        out_shape=[
            jax.ShapeDtypeStruct((N_NODES, S_DIM), jnp.float32),
            jax.ShapeDtypeStruct((N_NODES, S_DIM), jnp.float32),
        ],
    )(st, parts, wu, bu, wm, bm)


def _tc_fin_body(st_ref, p_ref, wu_ref, bu_ref, b_ref, wo_ref, bo_ref, out_ref, gs_ref):
    i = pl.program_id(0)

    @pl.when(i == 0)
    def _():
        gs_ref[...] = jnp.zeros_like(gs_ref)

    a = p_ref[0] + p_ref[1]
    st = st_ref[...] + jnp.maximum(
        jnp.dot(a, wu_ref[...], preferred_element_type=jnp.float32) + bu_ref[...],
        0.0,
    )
    b = b_ref[0, 0, :]
    onehot = (
        lax.broadcasted_iota(jnp.int32, (N_GRAPHS, BLK), 0) == b[None, :]
    ).astype(jnp.float32)
    gs_ref[...] += jnp.dot(onehot, st, preferred_element_type=jnp.float32)

    @pl.when(i == pl.num_programs(0) - 1)
    def _():
        out_ref[...] = (
            jnp.dot(gs_ref[...], wo_ref[...], preferred_element_type=jnp.float32)
            + bo_ref[...]
        )


def _tc_fin(st, parts, wu, bu, batch3, wo, bo):
    return pl.pallas_call(
        _tc_fin_body,
        grid=(N_BLKS,),
        in_specs=[
            pl.BlockSpec((BLK, S_DIM), lambda i: (i, 0)),
            pl.BlockSpec((NC, BLK, S_DIM), lambda i: (0, i, 0)),
            pl.BlockSpec((S_DIM, S_DIM), lambda i: (0, 0)),
            pl.BlockSpec((1, S_DIM), lambda i: (0, 0)),
            pl.BlockSpec((1, 1, BLK), lambda i: (i, 0, 0)),
            pl.BlockSpec((S_DIM, 1), lambda i: (0, 0)),
            pl.BlockSpec((1, 1), lambda i: (0, 0)),
        ],
        out_specs=pl.BlockSpec((N_GRAPHS, 1), lambda i: (0, 0)),
        out_shape=jax.ShapeDtypeStruct((N_GRAPHS, 1), jnp.float32),
        scratch_shapes=[pltpu.VMEM((N_GRAPHS, S_DIM), jnp.float32)],
    )(st, parts, wu, bu, batch3, wo, bo)


def kernel(x, edge_index, batch, Wi, bi, Wm, bm, Wu, bu, Wo, bo):
    pad = jnp.concatenate(
        [
            jnp.zeros((1, E_PAD - N_EDGES), jnp.int32),
            jnp.full((1, E_PAD - N_EDGES), N_NODES, jnp.int32),
        ],
        axis=0,
    )
    edge3 = jnp.concatenate([edge_index, pad], axis=1).reshape(2, N_CHUNKS, CHUNK)
    batch3 = batch.reshape(N_BLKS, 1, BLK)
    bi2 = bi.reshape(1, S_DIM)
    bo2 = bo.reshape(1, 1)

    st, msg = _tc_init(x, Wi, bi2, Wm[0], bm[0].reshape(1, S_DIM))
    for r in range(N_ROUNDS):
        parts = _sc_edge(msg, edge3)
        if r < N_ROUNDS - 1:
            st, msg = _tc_upd(
                st, parts,
                Wu[r], bu[r].reshape(1, S_DIM),
                Wm[r + 1], bm[r + 1].reshape(1, S_DIM),
            )
        else:
            out = _tc_fin(st, parts, Wu[r], bu[r].reshape(1, S_DIM), batch3, Wo, bo2)
    return out.reshape(-1)
